# Initial kernel scaffold; baseline (speedup 1.0000x reference)
import jax
import jax.numpy as jnp
from jax.experimental import pallas as pl


def kernel(adj):
    raise NotImplementedError("write your pallas kernel here")


# speedup vs baseline: 52.7658x; 52.7658x over previous
import functools

import jax
import jax.numpy as jnp
from jax import lax
from jax.experimental import pallas as pl
from jax.experimental.pallas import tpu as pltpu
from jax.experimental.pallas import tpu_sc as plsc

_TOPK = 4
_ROWS = 1024
_COLS = 32768
_NC = 2
_NS = 16
_L = 16
_NW = _NC * _NS
_RPW = _ROWS // _NW
_GROUPS = _RPW // 4
_UNROLL = 4
_STEPS = _COLS // (_L * _UNROLL)
_NEG_INF = float("-inf")


def _insert(v, idx, t0, t1, t2, t3, i0, i1, i2, i3):
    m0 = v > t0
    m1 = v > t1
    m2 = v > t2
    m3 = v > t3
    t3n = jnp.where(m2, t2, jnp.where(m3, v, t3))
    i3n = jnp.where(m2, i2, jnp.where(m3, idx, i3))
    t2n = jnp.where(m1, t1, jnp.where(m2, v, t2))
    i2n = jnp.where(m1, i1, jnp.where(m2, idx, i2))
    t1n = jnp.where(m0, t0, jnp.where(m1, v, t1))
    i1n = jnp.where(m0, i0, jnp.where(m1, idx, i1))
    t0n = jnp.where(m0, v, t0)
    i0n = jnp.where(m0, idx, i0)
    return t0n, t1n, t2n, t3n, i0n, i1n, i2n, i3n


def _scan_row(buf, iota):
    tneg = jnp.full((_L,), _NEG_INF, jnp.float32)
    izero = jnp.zeros((_L,), jnp.int32)

    def body(c, carry):
        t0, t1, t2, t3, i0, i1, i2, i3 = carry
        base = c * (_L * _UNROLL)
        for u in range(_UNROLL):
            off = base + u * _L
            v = buf[pl.ds(off, _L)]
            idx = iota + off
            t0, t1, t2, t3, i0, i1, i2, i3 = _insert(
                v, idx, t0, t1, t2, t3, i0, i1, i2, i3)
        return (t0, t1, t2, t3, i0, i1, i2, i3)

    init = (tneg, tneg, tneg, tneg, izero, izero, izero, izero)
    return lax.fori_loop(0, _STEPS, body, init)


def _merge_softmax(carry, lane_off, iota):
    ts = list(carry[:4])
    is_ = list(carry[4:])
    big = jnp.int32(2**30)
    gv, gi = [], []
    for _ in range(_TOPK):
        m = jnp.maximum(jnp.maximum(ts[0], ts[1]), jnp.maximum(ts[2], ts[3]))
        gmax = jnp.max(m)
        cand = [jnp.where(tj == gmax, ij, big) for tj, ij in zip(ts, is_)]
        mn = jnp.minimum(jnp.minimum(cand[0], cand[1]),
                         jnp.minimum(cand[2], cand[3]))
        gidx = jnp.min(mn)
        gv.append(gmax)
        gi.append(gidx)
        ts = [jnp.where(ij == gidx, _NEG_INF, tj) for tj, ij in zip(ts, is_)]
    dv = jnp.zeros((_L,), jnp.float32)
    iv = jnp.zeros((_L,), jnp.int32)
    for k in range(_TOPK):
        sel = iota == (lane_off + k)
        dv = jnp.where(sel, gv[k] - gv[0], dv)
        iv = jnp.where(sel, gi[k], iv)
    ev = jnp.exp(dv)
    in_row = (iota >= lane_off) & (iota < lane_off + _TOPK)
    ev = jnp.where(in_row, ev, 0.0)
    wv = ev / jnp.sum(ev)
    return wv, iv


def _make_kernel():
    mesh = plsc.VectorSubcoreMesh(core_axis_name="c", subcore_axis_name="s",
                                  num_cores=_NC, num_subcores=_NS)

    @functools.partial(
        pl.kernel,
        out_type=(
            jax.ShapeDtypeStruct((_ROWS * _TOPK,), jnp.float32),
            jax.ShapeDtypeStruct((_ROWS * _TOPK,), jnp.int32),
        ),
        mesh=mesh,
        scratch_types=(
            pltpu.VMEM((_COLS,), jnp.float32),
            pltpu.VMEM((_COLS,), jnp.float32),
            pltpu.VMEM((_RPW * _TOPK,), jnp.float32),
            pltpu.VMEM((_RPW * _TOPK,), jnp.int32),
            pltpu.SemaphoreType.DMA,
        ),
        compiler_params=pltpu.CompilerParams(needs_layout_passes=False),
    )
    def topk_route(adj_hbm, out_w_hbm, out_i_hbm, buf0, buf1, stw, sti, sem):
        cid = lax.axis_index("c")
        sid = lax.axis_index("s")
        wid = sid * _NC + cid
        row0 = wid * _RPW
        iota = lax.iota(jnp.int32, _L)

        def row_slice(r):
            return adj_hbm.at[pl.ds(r * _COLS, _COLS)]

        pltpu.sync_copy(row_slice(row0), buf0)

        def group(g, acc):
            r0 = row0 + 4 * g
            pltpu.async_copy(row_slice(r0 + 1), buf1, sem)
            w0, j0 = _merge_softmax(_scan_row(buf0, iota), 0, iota)
            pltpu.make_async_copy(row_slice(r0 + 1), buf1, sem).wait()

            pltpu.async_copy(row_slice(r0 + 2), buf0, sem)
            w1, j1 = _merge_softmax(_scan_row(buf1, iota), 4, iota)
            pltpu.make_async_copy(row_slice(r0 + 2), buf0, sem).wait()

            pltpu.async_copy(row_slice(r0 + 3), buf1, sem)
            w2, j2 = _merge_softmax(_scan_row(buf0, iota), 8, iota)
            pltpu.make_async_copy(row_slice(r0 + 3), buf1, sem).wait()

            @pl.when(g < _GROUPS - 1)
            def _():
                pltpu.async_copy(row_slice(r0 + 4), buf0, sem)

            w3, j3 = _merge_softmax(_scan_row(buf1, iota), 12, iota)

            @pl.when(g < _GROUPS - 1)
            def _():
                pltpu.make_async_copy(row_slice(r0 + 4), buf0, sem).wait()

            stw[pl.ds(g * _L, _L)] = w0 + w1 + w2 + w3
            sti[pl.ds(g * _L, _L)] = j0 + j1 + j2 + j3
            return acc

        lax.fori_loop(0, _GROUPS, group, jnp.int32(0))

        pltpu.sync_copy(stw, out_w_hbm.at[pl.ds(row0 * _TOPK, _RPW * _TOPK)])
        pltpu.sync_copy(sti, out_i_hbm.at[pl.ds(row0 * _TOPK, _RPW * _TOPK)])

    return topk_route


_topk_route = _make_kernel()


@jax.jit
def kernel(adj):
    b, h, n = adj.shape
    flat = adj.reshape(b * h * n)
    w, i = _topk_route(flat)
    return w.reshape(b, h, _TOPK), i.reshape(b, h, _TOPK)
